# baseline (speedup 1.0000x reference)
import functools

import jax
import jax.numpy as jnp
from jax.experimental import pallas as pl
from jax.experimental.pallas import tpu as pltpu


def _concat_kernel(tid_ref, p_hbm, x_ref, o_ref, sem, *, seq):
    b = pl.program_id(0)
    o_ref[0, :seq, :] = x_ref[0]
    pltpu.make_async_copy(
        p_hbm.at[pl.ds(tid_ref[b], 1), :, :],
        o_ref.at[pl.ds(0, 1), pl.ds(seq, 1), :],
        sem,
    ).start()
    pltpu.make_async_copy(
        p_hbm.at[pl.ds(tid_ref[b], 1), :, :],
        o_ref.at[pl.ds(0, 1), pl.ds(seq, 1), :],
        sem,
    ).wait()


def kernel(x, task_id, prompt):
    B, S, D = x.shape
    task_id32 = task_id.astype(jnp.int32)
    prompt3 = prompt.reshape(prompt.shape[0], 1, prompt.shape[1])

    out = pl.pallas_call(
        functools.partial(_concat_kernel, seq=S),
        grid=(B,),
        in_specs=[
            pl.BlockSpec(memory_space=pltpu.MemorySpace.SMEM),
            pl.BlockSpec(memory_space=pltpu.MemorySpace.HBM),
            pl.BlockSpec((1, S, D), lambda b: (b, 0, 0)),
        ],
        out_specs=pl.BlockSpec((1, S + 1, D), lambda b: (b, 0, 0)),
        out_shape=jax.ShapeDtypeStruct((B, S + 1, D), x.dtype),
        scratch_shapes=[pltpu.SemaphoreType.DMA],
    )(task_id32, prompt3, x)
    return (out, task_id)


# speedup vs baseline: 1.0229x; 1.0229x over previous
import jax
import jax.numpy as jnp
from jax.experimental import pallas as pl
from jax.experimental.pallas import tpu as pltpu


def _concat_kernel(task_id_ref, x_ref, p_ref, o_ref):
    seq = x_ref.shape[1]
    o_ref[0, :seq, :] = x_ref[0]
    o_ref[0, seq, :] = p_ref[0, 0]


def kernel(x, task_id, prompt):
    B, S, D = x.shape
    task_id32 = task_id.astype(jnp.int32)
    prompt3 = prompt.reshape(prompt.shape[0], 1, prompt.shape[1])

    grid_spec = pltpu.PrefetchScalarGridSpec(
        num_scalar_prefetch=1,
        grid=(B,),
        in_specs=[
            pl.BlockSpec((1, S, D), lambda b, tid: (b, 0, 0)),
            pl.BlockSpec((1, 1, D), lambda b, tid: (tid[b], 0, 0)),
        ],
        out_specs=pl.BlockSpec((1, S + 1, D), lambda b, tid: (b, 0, 0)),
    )

    out = pl.pallas_call(
        _concat_kernel,
        grid_spec=grid_spec,
        out_shape=jax.ShapeDtypeStruct((B, S + 1, D), x.dtype),
    )(task_id32, x, prompt3)
    return (out, task_id)
